# speedup vs baseline: 6.3067x; 6.3067x over previous
import functools

import jax
import jax.numpy as jnp
from jax import lax
from jax.experimental import pallas as pl
from jax.experimental.pallas import tpu as pltpu
from jax.experimental.pallas import tpu_sc as plsc

_NUM_WORKERS = 32
_CHUNK = 128


@functools.partial(jax.jit, static_argnums=(2, 3))
def _sc_gather(table, idx_flat, n, d):
    b_per_w = n // _NUM_WORKERS
    steps = b_per_w // _CHUNK
    mesh = plsc.VectorSubcoreMesh(core_axis_name="c", subcore_axis_name="s")

    @functools.partial(
        pl.kernel,
        mesh=mesh,
        out_type=jax.ShapeDtypeStruct((n, d), jnp.float32),
        scratch_types=[
            pltpu.VMEM((b_per_w,), jnp.int32),
            pltpu.VMEM((_CHUNK, d), jnp.float32),
            pltpu.SemaphoreType.DMA,
        ],
    )
    def body(table_hbm, idx_hbm, out_hbm, idx_v, rows_v, sem):
        wid = lax.axis_index("s") * 2 + lax.axis_index("c")
        base = pl.multiple_of(wid * b_per_w, _CHUNK)
        pltpu.sync_copy(idx_hbm.at[pl.ds(base, b_per_w)], idx_v)

        def step(g, carry):
            off = pl.multiple_of(g * _CHUNK, _CHUNK)
            pltpu.async_copy(
                table_hbm.at[idx_v.at[pl.ds(off, _CHUNK)]], rows_v, sem
            ).wait()
            pltpu.sync_copy(rows_v, out_hbm.at[pl.ds(base + off, _CHUNK)])
            return carry

        lax.fori_loop(0, steps, step, 0)

    return body(table, idx_flat)


def kernel(token_ids, embedding_matrix):
    b, t = token_ids.shape
    v, d = embedding_matrix.shape
    n = b * t
    idx_flat = token_ids.reshape(n).astype(jnp.int32)
    out = _sc_gather(embedding_matrix, idx_flat, n, d)
    return out.reshape(b, t, d)


# speedup vs baseline: 9.1212x; 1.4463x over previous
import functools

import jax
import jax.numpy as jnp
from jax import lax
from jax.experimental import pallas as pl
from jax.experimental.pallas import tpu as pltpu
from jax.experimental.pallas import tpu_sc as plsc

_NUM_WORKERS = 32
_CHUNK = 128
_K = 2
_SLAB = _K * _CHUNK
_NBUF = 2


@functools.partial(jax.jit, static_argnums=(2, 3))
def _sc_gather(table, idx_flat, n, d):
    b_per_w = n // _NUM_WORKERS
    steps = b_per_w // _SLAB
    mesh = plsc.VectorSubcoreMesh(core_axis_name="c", subcore_axis_name="s")

    @functools.partial(
        pl.kernel,
        mesh=mesh,
        out_type=jax.ShapeDtypeStruct((n, d), jnp.float32),
        scratch_types=[
            pltpu.VMEM((b_per_w,), jnp.int32),
            pltpu.VMEM((_SLAB, d), jnp.float32),
            pltpu.VMEM((_SLAB, d), jnp.float32),
            pltpu.SemaphoreType.DMA,
            pltpu.SemaphoreType.DMA,
            pltpu.SemaphoreType.DMA,
            pltpu.SemaphoreType.DMA,
        ],
    )
    def body(table_hbm, idx_hbm, out_hbm, idx_v, rows0, rows1, gs0, gs1, ws0, ws1):
        rows = (rows0, rows1)
        gsem = (gs0, gs1)
        wsem = (ws0, ws1)
        wid = lax.axis_index("s") * 2 + lax.axis_index("c")
        base = pl.multiple_of(wid * b_per_w, _SLAB)
        pltpu.sync_copy(idx_hbm.at[pl.ds(base, b_per_w)], idx_v)

        def start_gather(g, b):
            off = pl.multiple_of(g * _SLAB, _SLAB)
            for c in range(_K):
                pltpu.async_copy(
                    table_hbm.at[idx_v.at[pl.ds(off + c * _CHUNK, _CHUNK)]],
                    rows[b].at[pl.ds(c * _CHUNK, _CHUNK)],
                    gsem[b],
                )

        def wait_gather(b):
            pltpu.make_async_copy(
                table_hbm.at[pl.ds(0, _SLAB)], rows[b], gsem[b]
            ).wait()

        def start_write(g, b):
            off = pl.multiple_of(g * _SLAB, _SLAB)
            pltpu.async_copy(rows[b], out_hbm.at[pl.ds(base + off, _SLAB)], wsem[b])

        def wait_write(g, b):
            off = pl.multiple_of(g * _SLAB, _SLAB)
            pltpu.make_async_copy(
                rows[b], out_hbm.at[pl.ds(base + off, _SLAB)], wsem[b]
            ).wait()

        for b in range(_NBUF):
            start_gather(b, b)

        def outer(i, carry):
            for b in range(_NBUF):
                g = i * _NBUF + b
                wait_gather(b)
                start_write(g, b)
                wait_write(g, b)

                @pl.when(g + _NBUF < steps)
                def _():
                    start_gather(g + _NBUF, b)

            return carry

        lax.fori_loop(0, steps // _NBUF, outer, 0)

    return body(table, idx_flat)


def kernel(token_ids, embedding_matrix):
    b, t = token_ids.shape
    v, d = embedding_matrix.shape
    n = b * t
    idx_flat = token_ids.reshape(n).astype(jnp.int32)
    out = _sc_gather(embedding_matrix, idx_flat, n, d)
    return out.reshape(b, t, d)
